# Initial kernel scaffold; baseline (speedup 1.0000x reference)
import functools

import jax
import jax.numpy as jnp
from jax import lax
from jax.experimental import pallas as pl
from jax.experimental.pallas import tpu as pltpu
from jax.experimental.pallas import tpu_sc as plsc

N_NODES = 10000
D_FEAT = 128
N_EDGES = 320000

NC = 2
NS = 16
NW = NC * NS
EPW = N_EDGES // NW
K = 80
NCHUNK = EPW // K
RPT = N_NODES // NS
CW = 16


def _sc_body(x_hbm, row_hbm, col_hbm, psum_hbm, pcnt_hbm,
             acc, cnt, rows_v, row_v, col_v, ones_v, z128, z16, sem):
    cid = lax.axis_index("c")
    sid = lax.axis_index("s")
    wid = sid * NC + cid

    zero16 = jnp.zeros((16,), jnp.float32)
    one0 = jnp.where(lax.iota(jnp.int32, 16) == 0, 1.0, 0.0).astype(jnp.float32)

    def zrow(i, c):
        for j in range(D_FEAT // 16):
            z128[i, pl.ds(j * 16, 16)] = zero16
        return c
    lax.fori_loop(0, K, zrow, 0)

    def zcrow(i, c):
        z16[i, :] = zero16
        return c
    lax.fori_loop(0, RPT, zcrow, 0)

    def orow(i, c):
        ones_v[i, :] = one0
        return c
    lax.fori_loop(0, K, orow, 0)

    nbase = sid * RPT
    off = 0
    while off < RPT:
        n = min(K, RPT - off)
        pltpu.sync_copy(z128.at[pl.ds(0, n)], acc.at[pl.ds(nbase + off, n)])
        off += n
    pltpu.sync_copy(z16, cnt.at[pl.ds(nbase, RPT)])
    plsc.subcore_barrier()

    ebase = wid * EPW

    def chunk(ci, c):
        b = pl.multiple_of(ebase + ci * K, 8)
        pltpu.sync_copy(row_hbm.at[pl.ds(b, K)], row_v)
        pltpu.sync_copy(col_hbm.at[pl.ds(b, K)], col_v)
        pltpu.async_copy(x_hbm.at[row_v], rows_v, sem).wait()
        pltpu.sync_copy(rows_v, acc.at[col_v], add=True)
        pltpu.sync_copy(ones_v, cnt.at[col_v], add=True)
        return c
    lax.fori_loop(0, NCHUNK, chunk, 0)

    plsc.subcore_barrier()

    pltpu.sync_copy(acc.at[pl.ds(nbase, RPT)], psum_hbm.at[cid, pl.ds(nbase, RPT)])
    pltpu.sync_copy(cnt.at[pl.ds(nbase, RPT)], pcnt_hbm.at[cid, pl.ds(nbase, RPT)])


@jax.jit
def _aggregate(x, row, col):
    mesh = plsc.VectorSubcoreMesh(core_axis_name="c", subcore_axis_name="s")
    f = pl.kernel(
        _sc_body,
        out_type=[
            jax.ShapeDtypeStruct((NC, N_NODES, D_FEAT), jnp.float32),
            jax.ShapeDtypeStruct((NC, N_NODES, CW), jnp.float32),
        ],
        mesh=mesh,
        scratch_types=[
            pltpu.VMEM_SHARED((N_NODES, D_FEAT), jnp.float32),
            pltpu.VMEM_SHARED((N_NODES, CW), jnp.float32),
            pltpu.VMEM((K, D_FEAT), jnp.float32),
            pltpu.VMEM((K,), jnp.int32),
            pltpu.VMEM((K,), jnp.int32),
            pltpu.VMEM((K, CW), jnp.float32),
            pltpu.VMEM((K, D_FEAT), jnp.float32),
            pltpu.VMEM((RPT, CW), jnp.float32),
            pltpu.SemaphoreType.DMA,
        ],
    )
    return f(x, row, col)


def _dense_body(x_ref, p_ref, c_ref, w_ref, b_ref, o_ref):
    cntv = c_ref[0] + c_ref[1]
    cnt = jnp.sum(cntv, axis=1, keepdims=True)
    cnt = jnp.where(cnt == 0.0, 1.0, cnt)
    neigh = (p_ref[0] + p_ref[1]) / cnt
    w = w_ref[...]
    dn = (((1,), (1,)), ((), ()))
    acc = lax.dot_general(x_ref[...], w[:, :D_FEAT], dn,
                          preferred_element_type=jnp.float32,
                          precision=lax.Precision.HIGHEST)
    acc = acc + lax.dot_general(neigh, w[:, D_FEAT:], dn,
                                preferred_element_type=jnp.float32,
                                precision=lax.Precision.HIGHEST)
    o_ref[...] = jnp.maximum(acc + b_ref[...], 0.0)


@jax.jit
def _dense(x, psum, pcnt, W, b2):
    BM = 1000
    grid = (N_NODES // BM,)
    return pl.pallas_call(
        _dense_body,
        grid=grid,
        in_specs=[
            pl.BlockSpec((BM, D_FEAT), lambda i: (i, 0)),
            pl.BlockSpec((NC, BM, D_FEAT), lambda i: (0, i, 0)),
            pl.BlockSpec((NC, BM, CW), lambda i: (0, i, 0)),
            pl.BlockSpec((D_FEAT, 2 * D_FEAT), lambda i: (0, 0)),
            pl.BlockSpec((1, D_FEAT), lambda i: (0, 0)),
        ],
        out_specs=pl.BlockSpec((BM, D_FEAT), lambda i: (i, 0)),
        out_shape=jax.ShapeDtypeStruct((N_NODES, D_FEAT), jnp.float32),
    )(x, psum, pcnt, W, b2)


def kernel(x, edge_index, W, b):
    row = edge_index[0].astype(jnp.int32)
    col = edge_index[1].astype(jnp.int32)
    psum, pcnt = _aggregate(x, row, col)
    return _dense(x, psum, pcnt, W, b.reshape(1, -1))


# speedup vs baseline: 5.5161x; 5.5161x over previous
import functools

import jax
import jax.numpy as jnp
from jax import lax
from jax.experimental import pallas as pl
from jax.experimental.pallas import tpu as pltpu
from jax.experimental.pallas import tpu_sc as plsc

N_NODES = 10000
D_FEAT = 128
N_EDGES = 320000

NC = 2
NS = 16
NW = NC * NS
EPW = N_EDGES // NW
K = 80
NCHUNK = EPW // K
RPT = (N_NODES // NS) // 8 * 8
TAIL = N_NODES - NS * RPT
CW = 16


def _sc_body(x_hbm, row_hbm, col_hbm, psum_hbm, pcnt_hbm,
             acc, cnt, rows_v, row_v, col_v, ones_v, z128, z16, sem):
    cid = lax.axis_index("c")
    sid = lax.axis_index("s")
    wid = sid * NC + cid

    zero16 = jnp.zeros((16,), jnp.float32)
    one0 = jnp.where(lax.iota(jnp.int32, 16) == 0, 1.0, 0.0).astype(jnp.float32)

    def zrow(i, c):
        for j in range(D_FEAT // 16):
            z128[i, pl.ds(j * 16, 16)] = zero16
        return c
    lax.fori_loop(0, K, zrow, 0)

    def zcrow(i, c):
        z16[i, :] = zero16
        return c
    lax.fori_loop(0, RPT + TAIL, zcrow, 0)

    def orow(i, c):
        ones_v[i, :] = one0
        return c
    lax.fori_loop(0, K, orow, 0)

    nbase = sid * RPT

    def zero_rows(base, total):
        off = 0
        while off < total:
            n = min(K, total - off)
            pltpu.sync_copy(z128.at[pl.ds(0, n)], acc.at[pl.ds(base + off, n)])
            off += n

    zero_rows(nbase, RPT)
    pltpu.sync_copy(z16.at[pl.ds(0, RPT)], cnt.at[pl.ds(nbase, RPT)])

    @pl.when(sid == NS - 1)
    def _():
        zero_rows(NS * RPT, TAIL)
        pltpu.sync_copy(z16.at[pl.ds(0, TAIL)], cnt.at[pl.ds(NS * RPT, TAIL)])

    plsc.subcore_barrier()

    ebase = wid * EPW

    def chunk(ci, c):
        b = pl.multiple_of(ebase + ci * K, 8)
        pltpu.sync_copy(row_hbm.at[pl.ds(b, K)], row_v)
        pltpu.sync_copy(col_hbm.at[pl.ds(b, K)], col_v)
        pltpu.async_copy(x_hbm.at[row_v], rows_v, sem).wait()
        pltpu.sync_copy(rows_v, acc.at[col_v], add=True)
        pltpu.sync_copy(ones_v, cnt.at[col_v], add=True)
        return c
    lax.fori_loop(0, NCHUNK, chunk, 0)

    plsc.subcore_barrier()

    pltpu.sync_copy(acc.at[pl.ds(nbase, RPT)], psum_hbm.at[cid, pl.ds(nbase, RPT)])
    pltpu.sync_copy(cnt.at[pl.ds(nbase, RPT)], pcnt_hbm.at[cid, pl.ds(nbase, RPT)])

    @pl.when(sid == NS - 1)
    def _():
        base = NS * RPT
        pltpu.sync_copy(acc.at[pl.ds(base, TAIL)], psum_hbm.at[cid, pl.ds(base, TAIL)])
        pltpu.sync_copy(cnt.at[pl.ds(base, TAIL)], pcnt_hbm.at[cid, pl.ds(base, TAIL)])


@jax.jit
def _aggregate(x, row, col):
    mesh = plsc.VectorSubcoreMesh(core_axis_name="c", subcore_axis_name="s")
    f = pl.kernel(
        _sc_body,
        out_type=[
            jax.ShapeDtypeStruct((NC, N_NODES, D_FEAT), jnp.float32),
            jax.ShapeDtypeStruct((NC, N_NODES, CW), jnp.float32),
        ],
        mesh=mesh,
        scratch_types=[
            pltpu.VMEM_SHARED((N_NODES, D_FEAT), jnp.float32),
            pltpu.VMEM_SHARED((N_NODES, CW), jnp.float32),
            pltpu.VMEM((K, D_FEAT), jnp.float32),
            pltpu.VMEM((K,), jnp.int32),
            pltpu.VMEM((K,), jnp.int32),
            pltpu.VMEM((K, CW), jnp.float32),
            pltpu.VMEM((K, D_FEAT), jnp.float32),
            pltpu.VMEM((RPT + TAIL, CW), jnp.float32),
            pltpu.SemaphoreType.DMA,
        ],
        compiler_params=pltpu.CompilerParams(use_tc_tiling_on_sc=False),
    )
    return f(x, row, col)


def _dense_body(x_ref, p_ref, c_ref, w_ref, b_ref, o_ref):
    cntv = c_ref[0] + c_ref[1]
    cnt = jnp.sum(cntv, axis=1, keepdims=True)
    cnt = jnp.where(cnt == 0.0, 1.0, cnt)
    neigh = (p_ref[0] + p_ref[1]) / cnt
    w = w_ref[...]
    dn = (((1,), (1,)), ((), ()))
    acc = lax.dot_general(x_ref[...], w[:, :D_FEAT], dn,
                          preferred_element_type=jnp.float32,
                          precision=lax.Precision.HIGHEST)
    acc = acc + lax.dot_general(neigh, w[:, D_FEAT:], dn,
                                preferred_element_type=jnp.float32,
                                precision=lax.Precision.HIGHEST)
    o_ref[...] = jnp.maximum(acc + b_ref[...], 0.0)


@jax.jit
def _dense(x, psum, pcnt, W, b2):
    BM = 1000
    grid = (N_NODES // BM,)
    return pl.pallas_call(
        _dense_body,
        grid=grid,
        in_specs=[
            pl.BlockSpec((BM, D_FEAT), lambda i: (i, 0)),
            pl.BlockSpec((NC, BM, D_FEAT), lambda i: (0, i, 0)),
            pl.BlockSpec((NC, BM, CW), lambda i: (0, i, 0)),
            pl.BlockSpec((D_FEAT, 2 * D_FEAT), lambda i: (0, 0)),
            pl.BlockSpec((1, D_FEAT), lambda i: (0, 0)),
        ],
        out_specs=pl.BlockSpec((BM, D_FEAT), lambda i: (i, 0)),
        out_shape=jax.ShapeDtypeStruct((N_NODES, D_FEAT), jnp.float32),
    )(x, psum, pcnt, W, b2)


def kernel(x, edge_index, W, b):
    row = edge_index[0].astype(jnp.int32)
    col = edge_index[1].astype(jnp.int32)
    psum, pcnt = _aggregate(x, row, col)
    return _dense(x, psum, pcnt, W, b.reshape(1, -1))


# speedup vs baseline: 11.2973x; 2.0480x over previous
import functools

import jax
import jax.numpy as jnp
from jax import lax
from jax.experimental import pallas as pl
from jax.experimental.pallas import tpu as pltpu
from jax.experimental.pallas import tpu_sc as plsc

N_NODES = 10000
D_FEAT = 128
N_EDGES = 320000

NC = 2
NS = 16
NW = NC * NS
EPW = N_EDGES // NW
K = 80
NCHUNK = EPW // K
G = 25
NB = NCHUNK // G
RPT = (N_NODES // NS) // 8 * 8
TAIL = N_NODES - NS * RPT
CW = 16


def _sc_body(x_hbm, row_hbm, col_hbm, psum_hbm, pcnt_hbm,
             acc, cnt, rows0, rows1, rbuf0, rbuf1, cbuf0, cbuf1,
             ones_v, z16, sem0, sem1, isem):
    cid = lax.axis_index("c")
    sid = lax.axis_index("s")
    wid = sid * NC + cid

    zero16 = jnp.zeros((16,), jnp.float32)
    one0 = jnp.where(lax.iota(jnp.int32, 16) == 0, 1.0, 0.0).astype(jnp.float32)

    rbufs = (rbuf0, rbuf1)
    cbufs = (cbuf0, cbuf1)

    pltpu.async_copy(row_hbm.at[wid, 0], rbuf0, isem)
    pltpu.async_copy(col_hbm.at[wid, 0], cbuf0, isem)

    def zrow(i, c):
        for j in range(D_FEAT // 16):
            rows0[i, pl.ds(j * 16, 16)] = zero16
        return c
    lax.fori_loop(0, K, zrow, 0)

    def zcrow(i, c):
        z16[i, :] = zero16
        return c
    lax.fori_loop(0, K, zcrow, 0)

    def orow(i, c):
        ones_v[i, :] = one0
        return c
    lax.fori_loop(0, K, orow, 0)

    nbase = sid * RPT

    def zero_rows(base, total):
        off = 0
        while off < total:
            n = min(K, total - off)
            pltpu.sync_copy(rows0.at[pl.ds(0, n)], acc.at[pl.ds(base + off, n)])
            pltpu.sync_copy(z16.at[pl.ds(0, n)], cnt.at[pl.ds(base + off, n)])
            off += n

    zero_rows(nbase, RPT)

    @pl.when(sid == NS - 1)
    def _():
        zero_rows(NS * RPT, TAIL)

    pltpu.make_async_copy(row_hbm.at[wid, 0], rbuf0, isem).wait()
    pltpu.make_async_copy(col_hbm.at[wid, 0], cbuf0, isem).wait()
    plsc.subcore_barrier()

    def process_block(rb, cb):
        def gather(ci, buf, sem):
            pltpu.async_copy(x_hbm.at[rb.at[ci]], buf, sem)

        def drain_scatter(ci, buf, sem):
            pltpu.make_async_copy(x_hbm.at[rb.at[ci]], buf, sem).wait()
            pltpu.sync_copy(buf, acc.at[cb.at[ci]], add=True)
            pltpu.sync_copy(ones_v, cnt.at[cb.at[ci]], add=True)

        gather(0, rows0, sem0)

        def pair(i, c):
            ci0 = 2 * i
            gather(ci0 + 1, rows1, sem1)
            drain_scatter(ci0, rows0, sem0)
            gather(ci0 + 2, rows0, sem0)
            drain_scatter(ci0 + 1, rows1, sem1)
            return c
        lax.fori_loop(0, G // 2, pair, 0)
        drain_scatter(G - 1, rows0, sem0)

    for bi in range(NB):
        s = bi % 2
        if bi + 1 < NB:
            pltpu.async_copy(row_hbm.at[wid, bi + 1], rbufs[1 - s], isem)
            pltpu.async_copy(col_hbm.at[wid, bi + 1], cbufs[1 - s], isem)
        process_block(rbufs[s], cbufs[s])
        if bi + 1 < NB:
            pltpu.make_async_copy(row_hbm.at[wid, bi + 1], rbufs[1 - s], isem).wait()
            pltpu.make_async_copy(col_hbm.at[wid, bi + 1], cbufs[1 - s], isem).wait()

    plsc.subcore_barrier()

    pltpu.sync_copy(acc.at[pl.ds(nbase, RPT)], psum_hbm.at[cid, pl.ds(nbase, RPT)])
    pltpu.sync_copy(cnt.at[pl.ds(nbase, RPT)], pcnt_hbm.at[cid, pl.ds(nbase, RPT)])

    @pl.when(sid == NS - 1)
    def _():
        base = NS * RPT
        pltpu.sync_copy(acc.at[pl.ds(base, TAIL)], psum_hbm.at[cid, pl.ds(base, TAIL)])
        pltpu.sync_copy(cnt.at[pl.ds(base, TAIL)], pcnt_hbm.at[cid, pl.ds(base, TAIL)])


@jax.jit
def _aggregate(x, row, col):
    mesh = plsc.VectorSubcoreMesh(core_axis_name="c", subcore_axis_name="s")
    f = pl.kernel(
        _sc_body,
        out_type=[
            jax.ShapeDtypeStruct((NC, N_NODES, D_FEAT), jnp.float32),
            jax.ShapeDtypeStruct((NC, N_NODES, CW), jnp.float32),
        ],
        mesh=mesh,
        scratch_types=[
            pltpu.VMEM_SHARED((N_NODES, D_FEAT), jnp.float32),
            pltpu.VMEM_SHARED((N_NODES, CW), jnp.float32),
            pltpu.VMEM((K, D_FEAT), jnp.float32),
            pltpu.VMEM((K, D_FEAT), jnp.float32),
            pltpu.VMEM((G, K), jnp.int32),
            pltpu.VMEM((G, K), jnp.int32),
            pltpu.VMEM((G, K), jnp.int32),
            pltpu.VMEM((G, K), jnp.int32),
            pltpu.VMEM((K, CW), jnp.float32),
            pltpu.VMEM((K, CW), jnp.float32),
            pltpu.SemaphoreType.DMA,
            pltpu.SemaphoreType.DMA,
            pltpu.SemaphoreType.DMA,
        ],
        compiler_params=pltpu.CompilerParams(use_tc_tiling_on_sc=False),
    )
    return f(x, row, col)


def _dense_body(x_ref, p_ref, c_ref, w_ref, b_ref, o_ref):
    cntv = c_ref[0] + c_ref[1]
    cnt = jnp.sum(cntv, axis=1, keepdims=True)
    cnt = jnp.where(cnt == 0.0, 1.0, cnt)
    neigh = (p_ref[0] + p_ref[1]) / cnt
    w = w_ref[...]
    dn = (((1,), (1,)), ((), ()))
    acc = lax.dot_general(x_ref[...], w[:, :D_FEAT], dn,
                          preferred_element_type=jnp.float32,
                          precision=lax.Precision.HIGHEST)
    acc = acc + lax.dot_general(neigh, w[:, D_FEAT:], dn,
                                preferred_element_type=jnp.float32,
                                precision=lax.Precision.HIGHEST)
    o_ref[...] = jnp.maximum(acc + b_ref[...], 0.0)


@jax.jit
def _dense(x, psum, pcnt, W, b2):
    BM = 1000
    grid = (N_NODES // BM,)
    return pl.pallas_call(
        _dense_body,
        grid=grid,
        in_specs=[
            pl.BlockSpec((BM, D_FEAT), lambda i: (i, 0)),
            pl.BlockSpec((NC, BM, D_FEAT), lambda i: (0, i, 0)),
            pl.BlockSpec((NC, BM, CW), lambda i: (0, i, 0)),
            pl.BlockSpec((D_FEAT, 2 * D_FEAT), lambda i: (0, 0)),
            pl.BlockSpec((1, D_FEAT), lambda i: (0, 0)),
        ],
        out_specs=pl.BlockSpec((BM, D_FEAT), lambda i: (i, 0)),
        out_shape=jax.ShapeDtypeStruct((N_NODES, D_FEAT), jnp.float32),
    )(x, psum, pcnt, W, b2)


def kernel(x, edge_index, W, b):
    row = edge_index[0].astype(jnp.int32).reshape(NW, NB, G, K)
    col = edge_index[1].astype(jnp.int32).reshape(NW, NB, G, K)
    psum, pcnt = _aggregate(x, row, col)
    return _dense(x, psum, pcnt, W, b.reshape(1, -1))


# speedup vs baseline: 15.3855x; 1.3619x over previous
import jax
import jax.numpy as jnp
from jax import lax
from jax.experimental import pallas as pl
from jax.experimental.pallas import tpu as pltpu
from jax.experimental.pallas import tpu_sc as plsc

N_NODES = 10000
D_FEAT = 128
N_EDGES = 320000

NC = 2
NS = 16
NW = NC * NS
EPW = N_EDGES // NW
K = 80
NCHUNK = EPW // K
RPT = (N_NODES // NS) // 8 * 8
TAIL = N_NODES - NS * RPT
CW = 16


def _sc_body(x_hbm, edge_hbm, psum_hbm, pcnt_hbm,
             acc, cnt, rows0, rows1, rows2,
             ridx0, ridx1, ridx2, cidx0, cidx1, cidx2,
             cidx2nd0, cidx2nd1, cidx2nd2, ones_v, z16,
             gsem0, gsem1, gsem2, rsem0, rsem1, rsem2,
             csem0, csem1, csem2, ksem0, ksem1, ksem2):
    cid = lax.axis_index("c")
    sid = lax.axis_index("s")
    wid = sid * NC + cid
    ebase = wid * EPW

    rows = (rows0, rows1, rows2)
    ridx = (ridx0, ridx1, ridx2)
    cidx = (cidx0, cidx1, cidx2)
    cidx2nd = (cidx2nd0, cidx2nd1, cidx2nd2)
    gsem = (gsem0, gsem1, gsem2)
    rsem = (rsem0, rsem1, rsem2)
    csem = (csem0, csem1, csem2)
    ksem = (ksem0, ksem1, ksem2)

    def slab(c):
        return pl.ds(pl.multiple_of(ebase + c * K, 8), K)

    def issue_ridx(c, s):
        pltpu.async_copy(edge_hbm.at[0, slab(c)], ridx[s], rsem[s])

    def wait_ridx(c, s):
        pltpu.make_async_copy(edge_hbm.at[0, slab(c)], ridx[s], rsem[s]).wait()

    def issue_cidx(c, s):
        pltpu.async_copy(edge_hbm.at[1, slab(c)], cidx[s], ksem[s])

    def wait_cidx(c, s):
        pltpu.make_async_copy(edge_hbm.at[1, slab(c)], cidx[s], ksem[s]).wait()

    def issue_gather(s):
        pltpu.async_copy(x_hbm.at[ridx[s]], rows[s], gsem[s])

    def wait_gather(s):
        pltpu.make_async_copy(x_hbm.at[ridx[s]], rows[s], gsem[s]).wait()

    def wait_cnt(s):
        pltpu.make_async_copy(ones_v, cnt.at[cidx2nd[s]], csem[s]).wait()

    def do_cnt(s):
        for j in range(K // 16):
            cidx2nd[s][pl.ds(j * 16, 16)] = cidx[s][pl.ds(j * 16, 16)]
        pltpu.async_copy(ones_v, cnt.at[cidx2nd[s]], csem[s], add=True)

    issue_cidx(0, 0)
    issue_cidx(1, 1)
    issue_ridx(0, 0)
    issue_ridx(1, 1)
    issue_ridx(2, 2)

    zero16 = jnp.zeros((16,), jnp.float32)
    one0 = jnp.where(lax.iota(jnp.int32, 16) == 0, 1.0, 0.0).astype(jnp.float32)

    def zrow(i, c):
        for j in range(D_FEAT // 16):
            rows0[i, pl.ds(j * 16, 16)] = zero16
        return c
    lax.fori_loop(0, K, zrow, 0)

    def zcrow(i, c):
        z16[i, :] = zero16
        ones_v[i, :] = one0
        return c
    lax.fori_loop(0, K, zcrow, 0)

    nbase = sid * RPT

    def zero_rows(base, total):
        off = 0
        while off < total:
            n = min(K, total - off)
            pltpu.sync_copy(rows0.at[pl.ds(0, n)], acc.at[pl.ds(base + off, n)])
            pltpu.sync_copy(z16.at[pl.ds(0, n)], cnt.at[pl.ds(base + off, n)])
            off += n

    zero_rows(nbase, RPT)

    @pl.when(sid == NS - 1)
    def _():
        zero_rows(NS * RPT, TAIL)

    wait_ridx(0, 0)
    issue_gather(0)
    wait_ridx(1, 1)
    issue_gather(1)
    plsc.subcore_barrier()

    def phase(i, c, p):
        s0 = p % 3
        s1 = (p + 1) % 3
        s2 = (p + 2) % 3
        issue_cidx(c + 2, s2)
        wait_cidx(c, s0)
        wait_ridx(c + 2, s2)
        issue_gather(s2)
        wait_gather(s0)
        if p == 2:
            @pl.when(i < (NCHUNK - 5) // 3)
            def _():
                issue_ridx(c + 3, s0)
        else:
            issue_ridx(c + 3, s0)
        pltpu.sync_copy(rows[s0], acc.at[cidx[s0]], add=True)

        @pl.when(i > 0)
        def _():
            wait_cnt(s0)
        do_cnt(s0)

    def body(i, carry):
        phase(i, 3 * i, 0)
        phase(i, 3 * i + 1, 1)
        phase(i, 3 * i + 2, 2)
        return carry
    lax.fori_loop(0, (NCHUNK - 2) // 3, body, 0)

    for c in (NCHUNK - 2, NCHUNK - 1):
        s0 = c % 3
        wait_cidx(c, s0)
        wait_gather(s0)
        pltpu.sync_copy(rows[s0], acc.at[cidx[s0]], add=True)
        wait_cnt(s0)
        do_cnt(s0)

    for c in (NCHUNK - 3, NCHUNK - 2, NCHUNK - 1):
        wait_cnt(c % 3)

    plsc.subcore_barrier()

    pltpu.sync_copy(acc.at[pl.ds(nbase, RPT)], psum_hbm.at[cid, pl.ds(nbase, RPT)])
    pltpu.sync_copy(cnt.at[pl.ds(nbase, RPT)], pcnt_hbm.at[cid, pl.ds(nbase, RPT)])

    @pl.when(sid == NS - 1)
    def _():
        base = NS * RPT
        pltpu.sync_copy(acc.at[pl.ds(base, TAIL)], psum_hbm.at[cid, pl.ds(base, TAIL)])
        pltpu.sync_copy(cnt.at[pl.ds(base, TAIL)], pcnt_hbm.at[cid, pl.ds(base, TAIL)])


@jax.jit
def _aggregate(x, edges):
    mesh = plsc.VectorSubcoreMesh(core_axis_name="c", subcore_axis_name="s")
    f = pl.kernel(
        _sc_body,
        out_type=[
            jax.ShapeDtypeStruct((NC, N_NODES, D_FEAT), jnp.float32),
            jax.ShapeDtypeStruct((NC, N_NODES, CW), jnp.float32),
        ],
        mesh=mesh,
        scratch_types=[
            pltpu.VMEM_SHARED((N_NODES, D_FEAT), jnp.float32),
            pltpu.VMEM_SHARED((N_NODES, CW), jnp.float32),
            pltpu.VMEM((K, D_FEAT), jnp.float32),
            pltpu.VMEM((K, D_FEAT), jnp.float32),
            pltpu.VMEM((K, D_FEAT), jnp.float32),
            pltpu.VMEM((K,), jnp.int32),
            pltpu.VMEM((K,), jnp.int32),
            pltpu.VMEM((K,), jnp.int32),
            pltpu.VMEM((K,), jnp.int32),
            pltpu.VMEM((K,), jnp.int32),
            pltpu.VMEM((K,), jnp.int32),
            pltpu.VMEM((K,), jnp.int32),
            pltpu.VMEM((K,), jnp.int32),
            pltpu.VMEM((K,), jnp.int32),
            pltpu.VMEM((K, CW), jnp.float32),
            pltpu.VMEM((K, CW), jnp.float32),
        ] + [pltpu.SemaphoreType.DMA] * 12,
        compiler_params=pltpu.CompilerParams(use_tc_tiling_on_sc=False),
    )
    return f(x, edges)


def _dense_body(x_ref, p_ref, c_ref, w_ref, b_ref, o_ref):
    cntv = c_ref[0] + c_ref[1]
    cnt = jnp.sum(cntv, axis=1, keepdims=True)
    cnt = jnp.where(cnt == 0.0, 1.0, cnt)
    neigh = (p_ref[0] + p_ref[1]) / cnt
    w = w_ref[...]
    dn = (((1,), (1,)), ((), ()))
    acc = lax.dot_general(x_ref[...], w[:, :D_FEAT], dn,
                          preferred_element_type=jnp.float32)
    acc = acc + lax.dot_general(neigh, w[:, D_FEAT:], dn,
                                preferred_element_type=jnp.float32)
    o_ref[...] = jnp.maximum(acc + b_ref[...], 0.0)


@jax.jit
def _dense(x, psum, pcnt, W, b2):
    BM = 2000
    grid = (N_NODES // BM,)
    return pl.pallas_call(
        _dense_body,
        grid=grid,
        in_specs=[
            pl.BlockSpec((BM, D_FEAT), lambda i: (i, 0)),
            pl.BlockSpec((NC, BM, D_FEAT), lambda i: (0, i, 0)),
            pl.BlockSpec((NC, BM, CW), lambda i: (0, i, 0)),
            pl.BlockSpec((D_FEAT, 2 * D_FEAT), lambda i: (0, 0)),
            pl.BlockSpec((1, D_FEAT), lambda i: (0, 0)),
        ],
        out_specs=pl.BlockSpec((BM, D_FEAT), lambda i: (i, 0)),
        out_shape=jax.ShapeDtypeStruct((N_NODES, D_FEAT), jnp.float32),
    )(x, psum, pcnt, W, b2)


def kernel(x, edge_index, W, b):
    edges = edge_index.astype(jnp.int32)
    psum, pcnt = _aggregate(x, edges)
    return _dense(x, psum, pcnt, W, b.reshape(1, -1))


# speedup vs baseline: 15.9681x; 1.0379x over previous
import jax
import jax.numpy as jnp
from jax import lax
from jax.experimental import pallas as pl
from jax.experimental.pallas import tpu as pltpu
from jax.experimental.pallas import tpu_sc as plsc

N_NODES = 10000
D_FEAT = 128
N_EDGES = 320000

NC = 2
NS = 16
NW = NC * NS
EPW = N_EDGES // NW
K = 80
NCHUNK = EPW // K
RPT = (N_NODES // NS) // 8 * 8
TAIL = N_NODES - NS * RPT
CW = 16


def _sc_body(x_hbm, edge_hbm, psum_hbm, pcnt_hbm,
             acc, cnt, rows0, rows1, rows2,
             ridx0, ridx1, ridx2, cidx0, cidx1, cidx2,
             cidx2nd0, cidx2nd1, cidx2nd2, ones_v, z16,
             gsem0, gsem1, gsem2, rsem0, rsem1, rsem2,
             csem0, csem1, csem2, ksem0, ksem1, ksem2):
    cid = lax.axis_index("c")
    sid = lax.axis_index("s")
    wid = sid * NC + cid
    ebase = wid * EPW

    rows = (rows0, rows1, rows2)
    ridx = (ridx0, ridx1, ridx2)
    cidx = (cidx0, cidx1, cidx2)
    cidx2nd = (cidx2nd0, cidx2nd1, cidx2nd2)
    gsem = (gsem0, gsem1, gsem2)
    rsem = (rsem0, rsem1, rsem2)
    csem = (csem0, csem1, csem2)
    ksem = (ksem0, ksem1, ksem2)

    def slab(c):
        return pl.ds(pl.multiple_of(ebase + c * K, 8), K)

    def issue_ridx(c, s):
        pltpu.async_copy(edge_hbm.at[0, slab(c)], ridx[s], rsem[s])

    def wait_ridx(c, s):
        pltpu.make_async_copy(edge_hbm.at[0, slab(c)], ridx[s], rsem[s]).wait()

    def issue_cidx(c, s):
        pltpu.async_copy(edge_hbm.at[1, slab(c)], cidx[s], ksem[s])

    def wait_cidx(c, s):
        pltpu.make_async_copy(edge_hbm.at[1, slab(c)], cidx[s], ksem[s]).wait()

    def issue_gather(s):
        pltpu.async_copy(x_hbm.at[ridx[s]], rows[s], gsem[s])

    def wait_gather(s):
        pltpu.make_async_copy(x_hbm.at[ridx[s]], rows[s], gsem[s]).wait()

    def wait_cnt(s):
        pltpu.make_async_copy(ones_v, cnt.at[cidx2nd[s]], csem[s]).wait()

    def do_cnt(s):
        for j in range(K // 16):
            cidx2nd[s][pl.ds(j * 16, 16)] = cidx[s][pl.ds(j * 16, 16)]
        pltpu.async_copy(ones_v, cnt.at[cidx2nd[s]], csem[s], add=True)

    issue_cidx(0, 0)
    issue_cidx(1, 1)
    issue_ridx(0, 0)
    issue_ridx(1, 1)
    issue_ridx(2, 2)

    zero16 = jnp.zeros((16,), jnp.float32)
    one0 = jnp.where(lax.iota(jnp.int32, 16) == 0, 1.0, 0.0).astype(jnp.float32)

    def zrow(i, c):
        for j in range(D_FEAT // 16):
            rows0[i, pl.ds(j * 16, 16)] = zero16
        return c
    lax.fori_loop(0, K, zrow, 0)

    def zcrow(i, c):
        z16[i, :] = zero16
        ones_v[i, :] = one0
        return c
    lax.fori_loop(0, K, zcrow, 0)

    nbase = sid * RPT

    def zero_rows(base, total):
        off = 0
        while off < total:
            n = min(K, total - off)
            pltpu.sync_copy(rows0.at[pl.ds(0, n)], acc.at[pl.ds(base + off, n)])
            pltpu.sync_copy(z16.at[pl.ds(0, n)], cnt.at[pl.ds(base + off, n)])
            off += n

    zero_rows(nbase, RPT)

    @pl.when(sid == NS - 1)
    def _():
        zero_rows(NS * RPT, TAIL)

    wait_ridx(0, 0)
    issue_gather(0)
    wait_ridx(1, 1)
    issue_gather(1)
    plsc.subcore_barrier()

    def phase(i, c, p):
        s0 = p % 3
        s1 = (p + 1) % 3
        s2 = (p + 2) % 3
        issue_cidx(c + 2, s2)
        wait_cidx(c, s0)
        wait_ridx(c + 2, s2)
        issue_gather(s2)
        wait_gather(s0)
        if p == 2:
            @pl.when(i < (NCHUNK - 5) // 3)
            def _():
                issue_ridx(c + 3, s0)
        else:
            issue_ridx(c + 3, s0)
        pltpu.sync_copy(rows[s0], acc.at[cidx[s0]], add=True)

        @pl.when(i > 0)
        def _():
            wait_cnt(s0)
        do_cnt(s0)

    def body(i, carry):
        phase(i, 3 * i, 0)
        phase(i, 3 * i + 1, 1)
        phase(i, 3 * i + 2, 2)
        return carry
    lax.fori_loop(0, (NCHUNK - 2) // 3, body, 0)

    for c in (NCHUNK - 2, NCHUNK - 1):
        s0 = c % 3
        wait_cidx(c, s0)
        wait_gather(s0)
        pltpu.sync_copy(rows[s0], acc.at[cidx[s0]], add=True)
        wait_cnt(s0)
        do_cnt(s0)

    for c in (NCHUNK - 3, NCHUNK - 2, NCHUNK - 1):
        wait_cnt(c % 3)

    plsc.subcore_barrier()

    pltpu.sync_copy(acc.at[pl.ds(nbase, RPT)], psum_hbm.at[cid, pl.ds(nbase, RPT)])
    pltpu.sync_copy(cnt.at[pl.ds(nbase, RPT)],
                    pcnt_hbm.at[cid, pl.ds(nbase, RPT), pl.ds(0, CW)])

    @pl.when(sid == NS - 1)
    def _():
        base = NS * RPT
        pltpu.sync_copy(acc.at[pl.ds(base, TAIL)], psum_hbm.at[cid, pl.ds(base, TAIL)])
        pltpu.sync_copy(cnt.at[pl.ds(base, TAIL)],
                        pcnt_hbm.at[cid, pl.ds(base, TAIL), pl.ds(0, CW)])


@jax.jit
def _aggregate(x, edges):
    mesh = plsc.VectorSubcoreMesh(core_axis_name="c", subcore_axis_name="s")
    f = pl.kernel(
        _sc_body,
        out_type=[
            jax.ShapeDtypeStruct((NC, N_NODES, D_FEAT), jnp.float32),
            jax.ShapeDtypeStruct((NC, N_NODES, 128), jnp.float32),
        ],
        mesh=mesh,
        scratch_types=[
            pltpu.VMEM_SHARED((N_NODES, D_FEAT), jnp.float32),
            pltpu.VMEM_SHARED((N_NODES, CW), jnp.float32),
            pltpu.VMEM((K, D_FEAT), jnp.float32),
            pltpu.VMEM((K, D_FEAT), jnp.float32),
            pltpu.VMEM((K, D_FEAT), jnp.float32),
            pltpu.VMEM((K,), jnp.int32),
            pltpu.VMEM((K,), jnp.int32),
            pltpu.VMEM((K,), jnp.int32),
            pltpu.VMEM((K,), jnp.int32),
            pltpu.VMEM((K,), jnp.int32),
            pltpu.VMEM((K,), jnp.int32),
            pltpu.VMEM((K,), jnp.int32),
            pltpu.VMEM((K,), jnp.int32),
            pltpu.VMEM((K,), jnp.int32),
            pltpu.VMEM((K, CW), jnp.float32),
            pltpu.VMEM((K, CW), jnp.float32),
        ] + [pltpu.SemaphoreType.DMA] * 12,
        compiler_params=pltpu.CompilerParams(use_tc_tiling_on_sc=False),
    )
    return f(x, edges)


def _dense_body(x_ref, p_ref, c_ref, w_ref, b_ref, o_ref):
    cntv = c_ref[0, :, :CW] + c_ref[1, :, :CW]
    cnt = jnp.sum(cntv, axis=1, keepdims=True)
    cnt = jnp.where(cnt == 0.0, 1.0, cnt)
    neigh = (p_ref[0] + p_ref[1]) / cnt
    w = w_ref[...]
    dn = (((1,), (1,)), ((), ()))
    acc = lax.dot_general(x_ref[...], w[:, :D_FEAT], dn,
                          preferred_element_type=jnp.float32)
    acc = acc + lax.dot_general(neigh, w[:, D_FEAT:], dn,
                                preferred_element_type=jnp.float32)
    o_ref[...] = jnp.maximum(acc + b_ref[...], 0.0)


@jax.jit
def _dense(x, psum, pcnt, W, b2):
    BM = 2000
    grid = (N_NODES // BM,)
    return pl.pallas_call(
        _dense_body,
        grid=grid,
        in_specs=[
            pl.BlockSpec((BM, D_FEAT), lambda i: (i, 0)),
            pl.BlockSpec((NC, BM, D_FEAT), lambda i: (0, i, 0)),
            pl.BlockSpec((NC, BM, 128), lambda i: (0, i, 0)),
            pl.BlockSpec((D_FEAT, 2 * D_FEAT), lambda i: (0, 0)),
            pl.BlockSpec((1, D_FEAT), lambda i: (0, 0)),
        ],
        out_specs=pl.BlockSpec((BM, D_FEAT), lambda i: (i, 0)),
        out_shape=jax.ShapeDtypeStruct((N_NODES, D_FEAT), jnp.float32),
    )(x, psum, pcnt, W, b2)


def kernel(x, edge_index, W, b):
    edges = edge_index.astype(jnp.int32)
    psum, pcnt = _aggregate(x, edges)
    return _dense(x, psum, pcnt, W, b.reshape(1, -1))
